# baseline (speedup 1.0000x reference)
import functools

import jax
import jax.numpy as jnp
from jax import lax
from jax.experimental import pallas as pl
from jax.experimental.pallas import tpu as pltpu
from jax.experimental.pallas import tpu_sc as plsc

B = 16384
MF_D = 32
MLP_D = 64
H3 = 16
LW = 128

NC = 2
NS = 16
NW = NC * NS
BPW = B // NW
CH = 64
NCH = BPW // CH
NLV = CH // 16


def _sc_gather(user_ids, item_ids, mf_u_t, mf_i_t, ml_u_t, ml_i_t):
    mesh = plsc.VectorSubcoreMesh(core_axis_name="c", subcore_axis_name="s")

    @functools.partial(
        pl.kernel,
        mesh=mesh,
        out_type=[
            jax.ShapeDtypeStruct((B, LW), jnp.float32),
            jax.ShapeDtypeStruct((B, LW), jnp.float32),
            jax.ShapeDtypeStruct((B, LW), jnp.float32),
            jax.ShapeDtypeStruct((B, LW), jnp.float32),
        ],
        scratch_types=[
            pltpu.VMEM((NCH, CH), jnp.int32),
            pltpu.VMEM((NCH, CH), jnp.int32),
            pltpu.VMEM((NCH, CH), jnp.int32),
            pltpu.VMEM((NCH, CH), jnp.int32),
            pltpu.VMEM((2, CH, LW), jnp.float32),
            pltpu.VMEM((2, CH, LW), jnp.float32),
            pltpu.VMEM((2, CH, LW), jnp.float32),
            pltpu.VMEM((2, CH, LW), jnp.float32),
            pltpu.SemaphoreType.DMA,
        ],
    )
    def gather_kernel(uid_hbm, iid_hbm, mfu_t, mfi_t, mlu_t, mli_t,
                      mfu_o, mfi_o, mlu_o, mli_o,
                      umf, imf, uml, iml, mfu_v, mfi_v, mlu_v, mli_v, sem):
        wid = lax.axis_index("s") * NC + lax.axis_index("c")
        base = wid * BPW
        for c in range(NCH):
            pltpu.sync_copy(uid_hbm.at[pl.ds(base + c * CH, CH)], umf.at[c])
            pltpu.sync_copy(iid_hbm.at[pl.ds(base + c * CH, CH)], imf.at[c])
        for c in range(NCH):
            for v in range(NLV):
                sl = pl.ds(v * 16, 16)
                u = umf[c, sl]
                i = imf[c, sl]
                uml[c, sl] = u >> 1
                iml[c, sl] = i >> 1
                umf[c, sl] = u >> 2
                imf[c, sl] = i >> 2

        def fire(c):
            slot = c % 2
            return [
                pltpu.async_copy(mfu_t.at[umf.at[c]], mfu_v.at[slot], sem),
                pltpu.async_copy(mfi_t.at[imf.at[c]], mfi_v.at[slot], sem),
                pltpu.async_copy(mlu_t.at[uml.at[c]], mlu_v.at[slot], sem),
                pltpu.async_copy(mli_t.at[iml.at[c]], mli_v.at[slot], sem),
            ]

        def drain(c, handles):
            for h in handles:
                h.wait()
            slot = c % 2
            out_sl = pl.ds(base + c * CH, CH)
            pltpu.sync_copy(mfu_v.at[slot], mfu_o.at[out_sl])
            pltpu.sync_copy(mfi_v.at[slot], mfi_o.at[out_sl])
            pltpu.sync_copy(mlu_v.at[slot], mlu_o.at[out_sl])
            pltpu.sync_copy(mli_v.at[slot], mli_o.at[out_sl])

        prev = fire(0)
        for c in range(1, NCH):
            cur = fire(c)
            drain(c - 1, prev)
            prev = cur
        drain(NCH - 1, prev)

    return gather_kernel(user_ids, item_ids, mf_u_t, mf_i_t, ml_u_t, ml_i_t)


BS = 1024


def _tc_body(uid_r, iid_r, mfu_r, mfi_r, mlu_r, mli_r, w1u_r, w1i_r, b1_r,
             w2_r, b2_r, w3_r, b3_r, wmf_r, wh_r, bo_r, out_r):
    uid = uid_r[:]
    iid = iid_r[:]

    def sel2(x, s):
        return jnp.where((s & 1) == 1, x[:, MLP_D:], x[:, :MLP_D])

    def sel4(x, s):
        m = s & 3
        r = jnp.where(m == 0, x[:, 0:32], x[:, 32:64])
        r = jnp.where(m == 2, x[:, 64:96], r)
        return jnp.where(m == 3, x[:, 96:128], r)

    mlu = sel2(mlu_r[:], uid)
    mli = sel2(mli_r[:], iid)
    mf = sel4(mfu_r[:], uid) * sel4(mfi_r[:], iid)
    h = mlu @ w1u_r[:] + mli @ w1i_r[:] + b1_r[:]
    h = jnp.maximum(h, 0.0)
    h = jnp.maximum(h @ w2_r[:] + b2_r[:], 0.0)
    h = jnp.maximum(h @ w3_r[:] + b3_r[:], 0.0)
    s = jnp.sum(mf * wmf_r[:], axis=1) + jnp.sum(h * wh_r[:], axis=1) + bo_r[0, 0]
    out_r[:] = s


def _tc_mlp(uid, iid, mfu, mfi, mlu, mli, w1u, w1i, b1, w2, b2, w3, b3,
            wmf, wh, bo):
    grid = B // BS

    def batch_spec(d):
        return pl.BlockSpec((BS, d), lambda i: (i, 0))

    def full_spec(a, b):
        return pl.BlockSpec((a, b), lambda i: (0, 0))

    id_spec = pl.BlockSpec((BS, 1), lambda i: (i, 0))
    return pl.pallas_call(
        _tc_body,
        grid=(grid,),
        in_specs=[
            id_spec, id_spec,
            batch_spec(LW), batch_spec(LW), batch_spec(LW), batch_spec(LW),
            full_spec(MLP_D, MLP_D), full_spec(MLP_D, MLP_D), full_spec(1, MLP_D),
            full_spec(MLP_D, 32), full_spec(1, 32),
            full_spec(32, H3), full_spec(1, H3),
            full_spec(1, MF_D), full_spec(1, H3), full_spec(1, 1),
        ],
        out_specs=pl.BlockSpec((BS,), lambda i: (i,)),
        out_shape=jax.ShapeDtypeStruct((B,), jnp.float32),
    )(uid, iid, mfu, mfi, mlu, mli, w1u, w1i, b1, w2, b2, w3, b3, wmf, wh, bo)


def kernel(user_ids, item_ids, mf_user_table, mf_item_table, mlp_user_table,
           mlp_item_table, W1, b1, W2, b2, W3, b3, W_out, b_out):
    n_u, n_i = mf_user_table.shape[0], mf_item_table.shape[0]
    mfu, mfi, mlu, mli = _sc_gather(
        user_ids, item_ids,
        mf_user_table.reshape(n_u // 4, LW),
        mf_item_table.reshape(n_i // 4, LW),
        mlp_user_table.reshape(n_u // 2, LW),
        mlp_item_table.reshape(n_i // 2, LW))
    w1u = W1[:MLP_D]
    w1i = W1[MLP_D:]
    wmf = W_out[:MF_D, 0].reshape(1, MF_D)
    wh = W_out[MF_D:, 0].reshape(1, H3)
    return _tc_mlp(user_ids.reshape(B, 1), item_ids.reshape(B, 1), mfu, mfi, mlu, mli, w1u, w1i,
                   b1.reshape(1, MLP_D), W2, b2.reshape(1, 32), W3,
                   b3.reshape(1, H3), wmf, wh, b_out.reshape(1, 1))


# speedup vs baseline: 1.3705x; 1.3705x over previous
import functools

import jax
import jax.numpy as jnp
from jax import lax
from jax.experimental import pallas as pl
from jax.experimental.pallas import tpu as pltpu
from jax.experimental.pallas import tpu_sc as plsc

B = 16384
MF_D = 32
MLP_D = 64
H3 = 16
LW = 128
DUMP = B

NC = 2
NS = 16
NW = NC * NS
CW = 2944
CAP = 384

N_U = 100000
N_I = 1000000
RNG_U = N_U // NW
RNG_I = N_I // NW
CPW_U = (RNG_U + 127 + CW - 1) // CW
CPW_I = (RNG_I + 127 + CW - 1) // CW
MAXC_U = ((N_U - CW) // 128) * 128
MAXC_I = ((N_I - CW) // 128) * 128
N_AL_U = (N_U // 128) * 128
N_AL_I = (N_I // 128) * 128
TAIL_U = N_U - N_AL_U
TAIL_I = N_I - N_AL_I


def _sc_gather(user_ids, item_ids, mfu_t, mfi_t, mlu_t, mli_t,
               mfu_tl, mfi_tl, mlu_tl, mli_tl):
    mesh = plsc.VectorSubcoreMesh(core_axis_name="c", subcore_axis_name="s")

    @functools.partial(
        pl.kernel,
        mesh=mesh,
        compiler_params=pltpu.CompilerParams(needs_layout_passes=False),
        out_type=[
            jax.ShapeDtypeStruct((B + 64, LW), jnp.float32),
            jax.ShapeDtypeStruct((B + 64, LW), jnp.float32),
        ],
        scratch_types=[
            pltpu.VMEM((512,), jnp.int32),
            pltpu.VMEM((B + 64,), jnp.int32),
            pltpu.VMEM((2, 8, CW), jnp.float32),
            pltpu.VMEM((CAP,), jnp.int32),
            pltpu.VMEM((CAP,), jnp.int32),
            pltpu.VMEM((96, CAP + 1), jnp.float32),
            pltpu.VMEM((2, 64), jnp.int32),
            pltpu.VMEM((2, 64, LW), jnp.float32),
            pltpu.SemaphoreType.DMA,
            pltpu.SemaphoreType.DMA,
        ],
    )
    def gather_kernel(uid_h, iid_h, mfu_h, mfi_h, mlu_h, mli_h,
                      mfu_tl_h, mfi_tl_h, mlu_tl_h, mli_tl_h,
                      o_u, o_i,
                      idsb, l_pk, slab, wdr, wpos, fm, sidx, rm, ssem, csem):
        wid = lax.axis_index("s") * NC + lax.axis_index("c")
        iota16 = lax.iota(jnp.int32, 16)

        def build_list(ids_h, lo, hi, lo_al):
            def piece(q, off):
                pltpu.sync_copy(ids_h.at[pl.ds(q * 512, 512)], idsb)

                def body(g, off2):
                    ms, pks, css, pcs = [], [], [], []
                    for u in range(4):
                        sl = pl.ds((g * 4 + u) * 16, 16)
                        idv = idsb[sl]
                        m = jnp.logical_and(idv >= lo, idv < hi)
                        ms.append(m)
                        css.append(plsc.cumsum(m.astype(jnp.int32)))
                        pks.append(((idv - lo_al) << 14)
                                   | (iota16 + q * 512 + (g * 4 + u) * 16))
                        pcs.append(plsc.all_reduce_population_count(m)[0])
                    o = off2
                    for u in range(4):
                        plsc.store_scatter(l_pk, [css[u] - 1 + o], pks[u],
                                           mask=ms[u])
                        o = o + pcs[u]
                    return o

                return lax.fori_loop(0, 8, body, off)
            n = lax.fori_loop(0, 32, piece, 0)
            for t in range(4):
                l_pk[pl.ds(n + t * 16, 16)] = jnp.full((16,), -1, jnp.int32)
            return n

        def process_window(plan, win_lo, win_w, llen, out_ref):
            nv = (llen + 15) // 16

            def count_pass(g, n):
                pcs = []
                for u in range(4):
                    pk = l_pk[pl.ds((g * 4 + u) * 16, 16)]
                    arel = pk >> 14
                    m = jnp.logical_and(arel >= win_lo, arel < win_lo + win_w)
                    pcs.append(plsc.all_reduce_population_count(m)[0])
                return n + pcs[0] + pcs[1] + pcs[2] + pcs[3]

            m_c = lax.fori_loop(0, (nv + 3) // 4, count_pass, 0)

            def rnd(r, _):
                base = r * CAP

                def comp(g, off):
                    ms, ars, pss, css, pcs = [], [], [], [], []
                    for u in range(4):
                        pk = l_pk[pl.ds((g * 4 + u) * 16, 16)]
                        arel = pk >> 14
                        m = jnp.logical_and(arel >= win_lo,
                                            arel < win_lo + win_w)
                        ms.append(m)
                        ars.append(arel - win_lo)
                        pss.append(pk & 16383)
                        css.append(plsc.cumsum(m.astype(jnp.int32)))
                        pcs.append(plsc.all_reduce_population_count(m)[0])
                    o = off
                    for u in range(4):
                        rk = css[u] - 1 + o
                        msk = jnp.logical_and(
                            ms[u], jnp.logical_and(rk >= base,
                                                   rk < base + CAP))
                        plsc.store_scatter(wdr, [rk - base], ars[u],
                                           mask=msk)
                        plsc.store_scatter(wpos, [rk - base], pss[u],
                                           mask=msk)
                        o = o + pcs[u]
                    return o

                def prefill(g, _):
                    wpos[pl.ds(g * 16, 16)] = jnp.full((16,), DUMP, jnp.int32)
                    wdr[pl.ds(g * 16, 16)] = jnp.zeros((16,), jnp.int32)
                    return 0
                lax.fori_loop(0, CAP // 16, prefill, 0)
                lax.fori_loop(0, (nv + 3) // 4, comp, 0)
                m_r = jnp.clip(m_c - base, 0, CAP)
                ng = (m_r + 15) // 16

                plan[0][0](0)
                for i, (_issue, _wait, fmb) in enumerate(plan):
                    _wait()
                    if i + 1 < len(plan):
                        plan[i + 1][0]((i + 1) % 2)

                    def ext(g, _):
                        drv = wdr[pl.ds(g * 16, 16)]
                        for s8 in range(8):
                            fm[fmb + s8, pl.ds(g * 16, 16)] = \
                                plsc.load_gather(
                                    slab.at[i % 2],
                                    [jnp.full((16,), s8, jnp.int32), drv])
                        return 0

                    lax.fori_loop(0, ng, ext, 0)

                ng4 = (m_r + 63) // 64

                def group(g, _):
                    slot = g % 2

                    @pl.when(g >= 2)
                    def _drain():
                        pltpu.make_async_copy(
                            rm.at[0], out_ref.at[pl.ds(DUMP, 64)],
                            ssem).wait()

                    def tp(j2, _):
                        for k2 in range(6):
                            rm[slot, j2, pl.ds(k2 * 16, 16)] = \
                                plsc.load_gather(
                                    fm, [iota16 + k2 * 16,
                                         jnp.full((16,), 0, jnp.int32) +
                                         j2 + g * 64])
                        return 0
                    lax.fori_loop(0, 64, tp, 0)
                    for k4 in range(4):
                        sidx[slot, pl.ds(k4 * 16, 16)] = \
                            wpos[pl.ds(g * 64 + k4 * 16, 16)]
                    pltpu.async_copy(rm.at[slot],
                                     out_ref.at[sidx.at[slot]], ssem)
                    return 0

                lax.fori_loop(0, ng4, group, 0)

                def drain(_, __):
                    pltpu.make_async_copy(
                        rm.at[0], out_ref.at[pl.ds(DUMP, 64)], ssem).wait()
                    return 0
                lax.fori_loop(0, jnp.minimum(ng4, 2), drain, 0)
                return 0

            lax.fori_loop(0, (m_c + CAP - 1) // CAP, rnd, 0)

        def run_side(mf_h, mf_tl_h, ml_h, ml_tl_h, cpw, maxc, n_al,
                     tail_w, lo, llen, out_ref):
            lo_al = pl.multiple_of((lo // 128) * 128, 128)

            def chunk(c, _):
                s0 = pl.multiple_of(jnp.minimum(lo_al + c * CW, maxc), 128)

                def mk(tbl_h, p):
                    def issue(slot):
                        pltpu.async_copy(
                            tbl_h.at[pl.ds(p * 8, 8), pl.ds(s0, CW)],
                            slab.at[slot], csem)

                    def wait():
                        pltpu.make_async_copy(
                            tbl_h.at[pl.ds(0, 8), pl.ds(0, CW)],
                            slab.at[0], csem).wait()
                    return issue, wait

                plan = [mk(mf_h, p) + (p * 8,) for p in range(4)] + \
                       [mk(ml_h, p) + (32 + p * 8,) for p in range(8)]
                process_window(plan, s0 - lo_al, CW, llen, out_ref)
                return 0

            lax.fori_loop(0, cpw, chunk, 0)

            def mkt(tbl_h, p):
                def issue(slot):
                    pltpu.async_copy(tbl_h.at[pl.ds(p * 8, 8), :],
                                     slab.at[slot, :, pl.ds(0, 128)], csem)

                def wait():
                    pltpu.make_async_copy(
                        tbl_h.at[pl.ds(0, 8), :],
                        slab.at[0, :, pl.ds(0, 128)], csem).wait()
                return issue, wait

            planT = [mkt(mf_tl_h, p) + (p * 8,) for p in range(4)] + \
                    [mkt(ml_tl_h, p) + (32 + p * 8,) for p in range(8)]
            process_window(planT, n_al - lo_al, tail_w, llen, out_ref)

        lo_u = wid * RNG_U
        lo_u_al = (lo_u // 128) * 128
        llen_u = build_list(uid_h, lo_u, lo_u + RNG_U, lo_u_al)
        run_side(mfu_h, mfu_tl_h, mlu_h, mlu_tl_h, CPW_U, MAXC_U, N_AL_U,
                 TAIL_U, lo_u, llen_u, o_u)
        lo_i = wid * RNG_I
        lo_i_al = (lo_i // 128) * 128
        llen_i = build_list(iid_h, lo_i, lo_i + RNG_I, lo_i_al)
        run_side(mfi_h, mfi_tl_h, mli_h, mli_tl_h, CPW_I, MAXC_I, N_AL_I,
                 TAIL_I, lo_i, llen_i, o_i)

    return gather_kernel(user_ids, item_ids, mfu_t, mfi_t, mlu_t, mli_t,
                         mfu_tl, mfi_tl, mlu_tl, mli_tl)


BS = 1024


def _tc_body(u_r, i_r, w1u_r, w1i_r, b1_r, w2_r, b2_r,
             w3_r, b3_r, wmf_r, wh_r, bo_r, out_r):
    mf = u_r[:, :MF_D] * i_r[:, :MF_D]
    h = u_r[:, 32:96] @ w1u_r[:] + i_r[:, 32:96] @ w1i_r[:] + b1_r[:]
    h = jnp.maximum(h, 0.0)
    h = jnp.maximum(h @ w2_r[:] + b2_r[:], 0.0)
    h = jnp.maximum(h @ w3_r[:] + b3_r[:], 0.0)
    s = jnp.sum(mf * wmf_r[:], axis=1) + jnp.sum(h * wh_r[:], axis=1) \
        + bo_r[0, 0]
    out_r[:] = s


def _tc_mlp(pu, pi, w1u, w1i, b1, w2, b2, w3, b3, wmf, wh, bo):
    grid = B // BS

    def batch_spec():
        return pl.BlockSpec((BS, LW), lambda i: (i, 0))

    def full_spec(a, b):
        return pl.BlockSpec((a, b), lambda i: (0, 0))

    return pl.pallas_call(
        _tc_body,
        grid=(grid,),
        in_specs=[
            batch_spec(), batch_spec(),
            full_spec(MLP_D, MLP_D), full_spec(MLP_D, MLP_D),
            full_spec(1, MLP_D),
            full_spec(MLP_D, 32), full_spec(1, 32),
            full_spec(32, H3), full_spec(1, H3),
            full_spec(1, MF_D), full_spec(1, H3), full_spec(1, 1),
        ],
        out_specs=pl.BlockSpec((BS,), lambda i: (i,)),
        out_shape=jax.ShapeDtypeStruct((B,), jnp.float32),
    )(pu, pi, w1u, w1i, b1, w2, b2, w3, b3, wmf, wh, bo)


def _tail(table, n_al):
    t = table[n_al:, :]
    return jnp.pad(t, ((0, 128 - t.shape[0]), (0, 0))).T


def kernel(user_ids, item_ids, mf_user_table, mf_item_table, mlp_user_table,
           mlp_item_table, W1, b1, W2, b2, W3, b3, W_out, b_out):
    pu, pi = _sc_gather(
        user_ids, item_ids,
        mf_user_table.T, mf_item_table.T,
        mlp_user_table.T, mlp_item_table.T,
        _tail(mf_user_table, N_AL_U), _tail(mf_item_table, N_AL_I),
        _tail(mlp_user_table, N_AL_U), _tail(mlp_item_table, N_AL_I))
    w1u = W1[:MLP_D]
    w1i = W1[MLP_D:]
    wmf = W_out[:MF_D, 0].reshape(1, MF_D)
    wh = W_out[MF_D:, 0].reshape(1, H3)
    return _tc_mlp(pu, pi, w1u, w1i, b1.reshape(1, MLP_D),
                   W2, b2.reshape(1, 32), W3, b3.reshape(1, H3),
                   wmf, wh, b_out.reshape(1, 1))


# speedup vs baseline: 1.9432x; 1.4179x over previous
import functools

import jax
import jax.numpy as jnp
from jax import lax
from jax.experimental import pallas as pl
from jax.experimental.pallas import tpu as pltpu
from jax.experimental.pallas import tpu_sc as plsc

B = 16384
MF_D = 32
MLP_D = 64
H3 = 16
LW = 128
DUMP = B

NC = 2
NS = 16
NW = NC * NS
CW = 2944
CAP = 384

N_U = 100000
N_I = 1000000
RNG_U = N_U // NW
RNG_I = N_I // NW
CPW_U = (RNG_U + 127 + CW - 1) // CW
CPW_I = (RNG_I + 127 + CW - 1) // CW
MAXC_U = ((N_U - CW) // 128) * 128
MAXC_I = ((N_I - CW) // 128) * 128
N_AL_U = (N_U // 128) * 128
N_AL_I = (N_I // 128) * 128
TAIL_U = N_U - N_AL_U
TAIL_I = N_I - N_AL_I


def _sc_gather(user_ids, item_ids, mfu_t, mfi_t, mlu_t, mli_t,
               mfu_tl, mfi_tl, mlu_tl, mli_tl):
    mesh = plsc.VectorSubcoreMesh(core_axis_name="c", subcore_axis_name="s")

    @functools.partial(
        pl.kernel,
        mesh=mesh,
        compiler_params=pltpu.CompilerParams(needs_layout_passes=False),
        out_type=[
            jax.ShapeDtypeStruct((B + 64, LW), jnp.float32),
            jax.ShapeDtypeStruct((B + 64, LW), jnp.float32),
        ],
        scratch_types=[
            pltpu.VMEM((512,), jnp.int32),
            pltpu.VMEM((B + 64,), jnp.int32),
            pltpu.VMEM((2, 8, CW), jnp.float32),
            pltpu.VMEM((CAP,), jnp.int32),
            pltpu.VMEM((CAP,), jnp.int32),
            pltpu.VMEM((96, CAP + 1), jnp.float32),
            pltpu.VMEM((5, 16), jnp.int32),
            pltpu.VMEM((5, 16, LW), jnp.float32),
            pltpu.SemaphoreType.DMA,
            pltpu.SemaphoreType.DMA,
        ],
    )
    def gather_kernel(uid_h, iid_h, mfu_h, mfi_h, mlu_h, mli_h,
                      mfu_tl_h, mfi_tl_h, mlu_tl_h, mli_tl_h,
                      o_u, o_i,
                      idsb, l_pk, slab, wdr, wpos, fm, sidx, rm, ssem, csem):
        wid = lax.axis_index("s") * NC + lax.axis_index("c")
        iota16 = lax.iota(jnp.int32, 16)

        def build_list(ids_h, lo, hi, lo_al):
            def piece(q, off):
                pltpu.sync_copy(ids_h.at[pl.ds(q * 512, 512)], idsb)

                def body(g, off2):
                    ms, pks, css, pcs = [], [], [], []
                    for u in range(4):
                        sl = pl.ds((g * 4 + u) * 16, 16)
                        idv = idsb[sl]
                        m = jnp.logical_and(idv >= lo, idv < hi)
                        ms.append(m)
                        css.append(plsc.cumsum(m.astype(jnp.int32)))
                        pks.append(((idv - lo_al) << 14)
                                   | (iota16 + q * 512 + (g * 4 + u) * 16))
                        pcs.append(plsc.all_reduce_population_count(m)[0])
                    o = off2
                    for u in range(4):
                        plsc.store_scatter(l_pk, [css[u] - 1 + o], pks[u],
                                           mask=ms[u])
                        o = o + pcs[u]
                    return o

                return lax.fori_loop(0, 8, body, off)
            n = lax.fori_loop(0, 32, piece, 0)
            for t in range(4):
                l_pk[pl.ds(n + t * 16, 16)] = jnp.full((16,), -1, jnp.int32)
            return n

        def process_window(plan, win_lo, win_w, llen, out_ref):
            nv = (llen + 15) // 16

            def count_pass(g, n):
                pcs = []
                for u in range(4):
                    pk = l_pk[pl.ds((g * 4 + u) * 16, 16)]
                    arel = pk >> 14
                    m = jnp.logical_and(arel >= win_lo, arel < win_lo + win_w)
                    pcs.append(plsc.all_reduce_population_count(m)[0])
                return n + pcs[0] + pcs[1] + pcs[2] + pcs[3]

            m_c = lax.fori_loop(0, (nv + 3) // 4, count_pass, 0)

            def rnd(r, _):
                base = r * CAP

                def comp(g, off):
                    ms, ars, pss, css, pcs = [], [], [], [], []
                    for u in range(4):
                        pk = l_pk[pl.ds((g * 4 + u) * 16, 16)]
                        arel = pk >> 14
                        m = jnp.logical_and(arel >= win_lo,
                                            arel < win_lo + win_w)
                        ms.append(m)
                        ars.append(arel - win_lo)
                        pss.append(pk & 16383)
                        css.append(plsc.cumsum(m.astype(jnp.int32)))
                        pcs.append(plsc.all_reduce_population_count(m)[0])
                    o = off
                    for u in range(4):
                        rk = css[u] - 1 + o
                        msk = jnp.logical_and(
                            ms[u], jnp.logical_and(rk >= base,
                                                   rk < base + CAP))
                        plsc.store_scatter(wdr, [rk - base], ars[u],
                                           mask=msk)
                        plsc.store_scatter(wpos, [rk - base], pss[u],
                                           mask=msk)
                        o = o + pcs[u]
                    return o

                def prefill(g, _):
                    wpos[pl.ds(g * 16, 16)] = jnp.full((16,), DUMP, jnp.int32)
                    wdr[pl.ds(g * 16, 16)] = jnp.zeros((16,), jnp.int32)
                    return 0
                lax.fori_loop(0, CAP // 16, prefill, 0)
                lax.fori_loop(0, (nv + 3) // 4, comp, 0)
                m_r = jnp.clip(m_c - base, 0, CAP)
                ng = (m_r + 15) // 16

                plan[0][0](0)
                for i, (_issue, _wait, fmb) in enumerate(plan):
                    _wait()
                    if i + 1 < len(plan):
                        plan[i + 1][0]((i + 1) % 2)

                    def ext(g, _):
                        drv = wdr[pl.ds(g * 16, 16)]
                        for s8 in range(8):
                            fm[fmb + s8, pl.ds(g * 16, 16)] = \
                                plsc.load_gather(
                                    slab.at[i % 2],
                                    [jnp.full((16,), s8, jnp.int32), drv])
                        return 0

                    lax.fori_loop(0, ng, ext, 0)

                def group(g, _):
                    slot = g % 5

                    @pl.when(g >= 5)
                    def _drain():
                        pltpu.make_async_copy(
                            rm.at[0], out_ref.at[pl.ds(DUMP, 16)],
                            ssem).wait()

                    for j2 in range(16):
                        for k2 in range(6):
                            rm[slot, j2, pl.ds(k2 * 16, 16)] = \
                                plsc.load_gather(
                                    fm, [iota16 + k2 * 16,
                                         jnp.full((16,), j2, jnp.int32) +
                                         g * 16])
                    sidx[slot, :] = wpos[pl.ds(g * 16, 16)]
                    pltpu.async_copy(rm.at[slot],
                                     out_ref.at[sidx.at[slot]], ssem)
                    return 0

                lax.fori_loop(0, ng, group, 0)

                def drain(_, __):
                    pltpu.make_async_copy(
                        rm.at[0], out_ref.at[pl.ds(DUMP, 16)], ssem).wait()
                    return 0
                lax.fori_loop(0, jnp.minimum(ng, 5), drain, 0)
                return 0

            lax.fori_loop(0, (m_c + CAP - 1) // CAP, rnd, 0)

        def run_side(mf_h, mf_tl_h, ml_h, ml_tl_h, cpw, maxc, n_al,
                     tail_w, lo, llen, out_ref):
            lo_al = pl.multiple_of((lo // 128) * 128, 128)

            def chunk(c, _):
                s0 = pl.multiple_of(jnp.minimum(lo_al + c * CW, maxc), 128)

                def mk(tbl_h, p):
                    def issue(slot):
                        pltpu.async_copy(
                            tbl_h.at[pl.ds(p * 8, 8), pl.ds(s0, CW)],
                            slab.at[slot], csem)

                    def wait():
                        pltpu.make_async_copy(
                            tbl_h.at[pl.ds(0, 8), pl.ds(0, CW)],
                            slab.at[0], csem).wait()
                    return issue, wait

                plan = [mk(mf_h, p) + (p * 8,) for p in range(4)] + \
                       [mk(ml_h, p) + (32 + p * 8,) for p in range(8)]
                process_window(plan, s0 - lo_al, CW, llen, out_ref)
                return 0

            lax.fori_loop(0, cpw, chunk, 0)

            def mkt(tbl_h, p):
                def issue(slot):
                    pltpu.async_copy(tbl_h.at[pl.ds(p * 8, 8), :],
                                     slab.at[slot, :, pl.ds(0, 128)], csem)

                def wait():
                    pltpu.make_async_copy(
                        tbl_h.at[pl.ds(0, 8), :],
                        slab.at[0, :, pl.ds(0, 128)], csem).wait()
                return issue, wait

            planT = [mkt(mf_tl_h, p) + (p * 8,) for p in range(4)] + \
                    [mkt(ml_tl_h, p) + (32 + p * 8,) for p in range(8)]
            process_window(planT, n_al - lo_al, tail_w, llen, out_ref)

        lo_u = wid * RNG_U
        lo_u_al = (lo_u // 128) * 128
        llen_u = build_list(uid_h, lo_u, lo_u + RNG_U, lo_u_al)
        run_side(mfu_h, mfu_tl_h, mlu_h, mlu_tl_h, CPW_U, MAXC_U, N_AL_U,
                 TAIL_U, lo_u, llen_u, o_u)
        lo_i = wid * RNG_I
        lo_i_al = (lo_i // 128) * 128
        llen_i = build_list(iid_h, lo_i, lo_i + RNG_I, lo_i_al)
        run_side(mfi_h, mfi_tl_h, mli_h, mli_tl_h, CPW_I, MAXC_I, N_AL_I,
                 TAIL_I, lo_i, llen_i, o_i)

    return gather_kernel(user_ids, item_ids, mfu_t, mfi_t, mlu_t, mli_t,
                         mfu_tl, mfi_tl, mlu_tl, mli_tl)


BS = 1024


def _tc_body(u_r, i_r, w1u_r, w1i_r, b1_r, w2_r, b2_r,
             w3_r, b3_r, wmf_r, wh_r, bo_r, out_r):
    mf = u_r[:, :MF_D] * i_r[:, :MF_D]
    h = u_r[:, 32:96] @ w1u_r[:] + i_r[:, 32:96] @ w1i_r[:] + b1_r[:]
    h = jnp.maximum(h, 0.0)
    h = jnp.maximum(h @ w2_r[:] + b2_r[:], 0.0)
    h = jnp.maximum(h @ w3_r[:] + b3_r[:], 0.0)
    s = jnp.sum(mf * wmf_r[:], axis=1) + jnp.sum(h * wh_r[:], axis=1) \
        + bo_r[0, 0]
    out_r[:] = s


def _tc_mlp(pu, pi, w1u, w1i, b1, w2, b2, w3, b3, wmf, wh, bo):
    grid = B // BS

    def batch_spec():
        return pl.BlockSpec((BS, LW), lambda i: (i, 0))

    def full_spec(a, b):
        return pl.BlockSpec((a, b), lambda i: (0, 0))

    return pl.pallas_call(
        _tc_body,
        grid=(grid,),
        in_specs=[
            batch_spec(), batch_spec(),
            full_spec(MLP_D, MLP_D), full_spec(MLP_D, MLP_D),
            full_spec(1, MLP_D),
            full_spec(MLP_D, 32), full_spec(1, 32),
            full_spec(32, H3), full_spec(1, H3),
            full_spec(1, MF_D), full_spec(1, H3), full_spec(1, 1),
        ],
        out_specs=pl.BlockSpec((BS,), lambda i: (i,)),
        out_shape=jax.ShapeDtypeStruct((B,), jnp.float32),
    )(pu, pi, w1u, w1i, b1, w2, b2, w3, b3, wmf, wh, bo)


def _tail(table, n_al):
    t = table[n_al:, :]
    return jnp.pad(t, ((0, 128 - t.shape[0]), (0, 0))).T


def kernel(user_ids, item_ids, mf_user_table, mf_item_table, mlp_user_table,
           mlp_item_table, W1, b1, W2, b2, W3, b3, W_out, b_out):
    pu, pi = _sc_gather(
        user_ids, item_ids,
        mf_user_table.T, mf_item_table.T,
        mlp_user_table.T, mlp_item_table.T,
        _tail(mf_user_table, N_AL_U), _tail(mf_item_table, N_AL_I),
        _tail(mlp_user_table, N_AL_U), _tail(mlp_item_table, N_AL_I))
    w1u = W1[:MLP_D]
    w1i = W1[MLP_D:]
    wmf = W_out[:MF_D, 0].reshape(1, MF_D)
    wh = W_out[MF_D:, 0].reshape(1, H3)
    return _tc_mlp(pu, pi, w1u, w1i, b1.reshape(1, MLP_D),
                   W2, b2.reshape(1, 32), W3, b3.reshape(1, H3),
                   wmf, wh, b_out.reshape(1, 1))
